# Initial kernel scaffold; baseline (speedup 1.0000x reference)
import jax
import jax.numpy as jnp
from jax.experimental import pallas as pl


def kernel(M, Wq, Wk, Wv, W_out, b_out, gate):
    raise NotImplementedError("write your pallas kernel here")


# speedup vs baseline: 9.8744x; 9.8744x over previous
import functools

import jax
import jax.numpy as jnp
from jax.experimental import pallas as pl
from jax.experimental.pallas import tpu as pltpu

_H = 8
_TOPK = 16


def _topk_mask(s, k):
    n = s.shape[-1]
    iota = jax.lax.broadcasted_iota(jnp.int32, s.shape, 1)
    work = s
    mask = jnp.zeros(s.shape, jnp.float32)
    for _ in range(k):
        m = jnp.max(work, axis=-1, keepdims=True)
        idx = jnp.min(jnp.where(work == m, iota, n), axis=-1, keepdims=True)
        hit = iota == idx
        mask = jnp.where(hit, 1.0, mask)
        work = jnp.where(hit, -jnp.inf, work)
    return mask


def _dot(a, b, dn):
    return jax.lax.dot_general(a, b, dn, preferred_element_type=jnp.float32)


_MM = (((1,), (0,)), ((), ()))
_NT = (((1,), (1,)), ((), ()))


def _body(m_ref, wq_ref, wk_ref, wv_ref, wo_ref, bo_ref, gate_ref,
          mt_ref, a_ref, ssum_ref, dacc_ref, *, scale):
    h = pl.program_id(1)
    mb = m_ref[0]
    q = _dot(mb, wq_ref[0], _MM)
    kk = _dot(mb, wk_ref[0], _MM)
    v = _dot(mb, wv_ref[0], _MM)
    s = _dot(q, kk, _NT) * scale

    @pl.when(h == 0)
    def _():
        ssum_ref[...] = s

    @pl.when(h != 0)
    def _():
        ssum_ref[...] = ssum_ref[...] + s

    mask = _topk_mask(s, _TOPK)
    m1 = jnp.max(s, axis=-1, keepdims=True)
    p = jnp.where(mask == 1.0, jnp.exp(s - m1), 0.0)
    attn = p / jnp.sum(p, axis=-1, keepdims=True)
    o = _dot(attn, v, _MM)
    dpart = _dot(o, wo_ref[...], _MM)

    @pl.when(h == 0)
    def _():
        dacc_ref[...] = dpart

    @pl.when(h != 0)
    def _():
        dacc_ref[...] = dacc_ref[...] + dpart

    @pl.when(h == _H - 1)
    def _():
        delta = dacc_ref[...] + bo_ref[...]
        mt_ref[...] = (mb + gate_ref[...] * delta)[None]
        avg = ssum_ref[...] * jnp.float32(1.0 / _H)
        a_ref[...] = _topk_mask(avg, _TOPK)[None]


def kernel(M, Wq, Wk, Wv, W_out, b_out, gate):
    B, C, NP = M.shape
    DM = Wq.shape[1]
    d = DM // _H
    scale = d ** (-0.5)
    bo = jnp.broadcast_to(b_out.reshape(1, NP), (1, NP))
    gt = jnp.broadcast_to(jnp.asarray(gate, jnp.float32).reshape(1, 1), (1, NP))
    wq_h = Wq.reshape(NP, _H, d).transpose(1, 0, 2)
    wk_h = Wk.reshape(NP, _H, d).transpose(1, 0, 2)
    wv_h = Wv.reshape(NP, _H, d).transpose(1, 0, 2)

    mt, a = pl.pallas_call(
        functools.partial(_body, scale=scale),
        grid=(B, _H),
        in_specs=[
            pl.BlockSpec((1, C, NP), lambda b, h: (b, 0, 0)),
            pl.BlockSpec((1, NP, d), lambda b, h: (h, 0, 0)),
            pl.BlockSpec((1, NP, d), lambda b, h: (h, 0, 0)),
            pl.BlockSpec((1, NP, d), lambda b, h: (h, 0, 0)),
            pl.BlockSpec((d, NP), lambda b, h: (h, 0)),
            pl.BlockSpec((1, NP), lambda b, h: (0, 0)),
            pl.BlockSpec((1, NP), lambda b, h: (0, 0)),
        ],
        out_specs=[
            pl.BlockSpec((1, C, NP), lambda b, h: (b, 0, 0)),
            pl.BlockSpec((1, C, C), lambda b, h: (b, 0, 0)),
        ],
        out_shape=[
            jax.ShapeDtypeStruct((B, C, NP), jnp.float32),
            jax.ShapeDtypeStruct((B, C, C), jnp.float32),
        ],
        scratch_shapes=[
            pltpu.VMEM((C, C), jnp.float32),
            pltpu.VMEM((C, NP), jnp.float32),
        ],
        compiler_params=pltpu.CompilerParams(
            dimension_semantics=("parallel", "arbitrary"),
        ),
    )(M, wq_h, wk_h, wv_h, W_out, bo, gt)
    return (mt, a)


# speedup vs baseline: 23.2881x; 2.3584x over previous
import functools

import jax
import jax.numpy as jnp
from jax.experimental import pallas as pl
from jax.experimental.pallas import tpu as pltpu

_H = 8
_TOPK = 16


def _dot(a, b, dn):
    return jax.lax.dot_general(a, b, dn, preferred_element_type=jnp.float32)


_MM = (((1,), (0,)), ((), ()))
_NT = (((1,), (1,)), ((), ()))


def _topk_threshold(s, k):
    m1 = jnp.max(s, axis=-1, keepdims=True)
    work, m = s, m1
    for _ in range(k - 1):
        work = jnp.where(work == m, -jnp.inf, work)
        m = jnp.max(work, axis=-1, keepdims=True)
    return m1, m


def _topk_mask_exact(s, k):
    n = s.shape[-1]
    r = s.shape[0]
    work = s
    cum = jnp.zeros((r, 1), jnp.float32)
    t = jnp.zeros((r, 1), jnp.float32)
    prevc = jnp.zeros((r, 1), jnp.float32)
    kf = jnp.float32(k)
    for _ in range(k):
        m = jnp.max(work, axis=-1, keepdims=True)
        eq = work == m
        cnt = jnp.sum(jnp.where(eq, 1.0, 0.0), axis=-1, keepdims=True)
        newcum = cum + cnt
        crossed = (cum < kf) & (newcum >= kf)
        t = jnp.where(crossed, m, t)
        prevc = jnp.where(crossed, cum, prevc)
        cum = newcum
        work = jnp.where(eq, -jnp.inf, work)
    need = kf - prevc
    eqt = s == t
    ioe = jax.lax.broadcasted_iota(jnp.int32, (n, n), 0)
    ioc = jax.lax.broadcasted_iota(jnp.int32, (n, n), 1)
    ltri = jnp.where(ioe < ioc, 1.0, 0.0)
    rank = _dot(jnp.where(eqt, 1.0, 0.0), ltri, _MM)
    return jnp.where((s > t) | (eqt & (rank < need)), 1.0, 0.0)


def _body(m_ref, wqh_ref, wkh_ref, wvh_ref, wq_ref, wk_ref, woh_ref,
          bo_ref, gate_ref, mt_ref, a_ref, dacc_ref, *, scale):
    h = pl.program_id(1)
    mb = m_ref[0]
    q = _dot(mb, wqh_ref[0], _MM)
    kk = _dot(mb, wkh_ref[0], _MM)
    v = _dot(mb, wvh_ref[0], _MM)
    s = _dot(q, kk, _NT)

    m1, t = _topk_threshold(s, _TOPK)
    p = jnp.where(s >= t, jnp.exp((s - m1) * scale), 0.0)
    denom = jnp.sum(p, axis=-1, keepdims=True)
    o = _dot(p, v, _MM) / denom
    dpart = _dot(o, woh_ref[0], _MM)

    @pl.when(h == 0)
    def _():
        dacc_ref[...] = dpart

    @pl.when(h != 0)
    def _():
        dacc_ref[...] = dacc_ref[...] + dpart

    @pl.when(h == _H - 1)
    def _():
        delta = dacc_ref[...] + bo_ref[...]
        mt_ref[...] = (mb + gate_ref[...] * delta)[None]
        qf = _dot(mb, wq_ref[...], _MM)
        kf = _dot(mb, wk_ref[...], _MM)
        savg = _dot(qf, kf, _NT)
        a_ref[...] = _topk_mask_exact(savg, _TOPK)[None]


def kernel(M, Wq, Wk, Wv, W_out, b_out, gate):
    B, C, NP = M.shape
    DM = Wq.shape[1]
    d = DM // _H
    scale = d ** (-0.5)
    bo = b_out.reshape(1, NP)
    gt = jnp.broadcast_to(jnp.asarray(gate, jnp.float32).reshape(1, 1), (1, NP))
    wq_h = Wq.reshape(NP, _H, d).transpose(1, 0, 2)
    wk_h = Wk.reshape(NP, _H, d).transpose(1, 0, 2)
    wv_h = Wv.reshape(NP, _H, d).transpose(1, 0, 2)
    wo_h = W_out.reshape(_H, d, NP)

    mt, a = pl.pallas_call(
        functools.partial(_body, scale=scale),
        grid=(B, _H),
        in_specs=[
            pl.BlockSpec((1, C, NP), lambda b, h: (b, 0, 0)),
            pl.BlockSpec((1, NP, d), lambda b, h: (h, 0, 0)),
            pl.BlockSpec((1, NP, d), lambda b, h: (h, 0, 0)),
            pl.BlockSpec((1, NP, d), lambda b, h: (h, 0, 0)),
            pl.BlockSpec((NP, DM), lambda b, h: (0, 0)),
            pl.BlockSpec((NP, DM), lambda b, h: (0, 0)),
            pl.BlockSpec((1, d, NP), lambda b, h: (h, 0, 0)),
            pl.BlockSpec((1, NP), lambda b, h: (0, 0)),
            pl.BlockSpec((1, NP), lambda b, h: (0, 0)),
        ],
        out_specs=[
            pl.BlockSpec((1, C, NP), lambda b, h: (b, 0, 0)),
            pl.BlockSpec((1, C, C), lambda b, h: (b, 0, 0)),
        ],
        out_shape=[
            jax.ShapeDtypeStruct((B, C, NP), jnp.float32),
            jax.ShapeDtypeStruct((B, C, C), jnp.float32),
        ],
        scratch_shapes=[
            pltpu.VMEM((C, NP), jnp.float32),
        ],
        compiler_params=pltpu.CompilerParams(
            dimension_semantics=("parallel", "arbitrary"),
        ),
    )(M, wq_h, wk_h, wv_h, Wq, Wk, wo_h, bo, gt)
    return (mt, a)
